# Initial kernel scaffold; baseline (speedup 1.0000x reference)
import jax
import jax.numpy as jnp
from jax.experimental import pallas as pl


def kernel(x):
    raise NotImplementedError("write your pallas kernel here")


# speedup vs baseline: 41.2062x; 41.2062x over previous
import functools

import jax
import jax.numpy as jnp
from jax import lax
from jax.experimental import pallas as pl
from jax.experimental.pallas import tpu as pltpu
from jax.experimental.pallas import tpu_sc as plsc

_NUM_BINS = 2048
_L = 16
_NC = 2
_NS = 16
_NW = _NC * _NS


def _absmax_body(x_ref, out_ref):
    i = pl.program_id(0)
    m = jnp.max(jnp.abs(x_ref[...]))

    @pl.when(i == 0)
    def _():
        out_ref[0, 0] = m

    @pl.when(i != 0)
    def _():
        out_ref[0, 0] = jnp.maximum(out_ref[0, 0], m)


def _absmax(x2d):
    rows, cols = x2d.shape
    block_rows = 512
    return pl.pallas_call(
        _absmax_body,
        grid=(rows // block_rows,),
        in_specs=[pl.BlockSpec((block_rows, cols), lambda i: (i, 0))],
        out_specs=pl.BlockSpec(memory_space=pltpu.SMEM),
        out_shape=jax.ShapeDtypeStruct((1, 1), jnp.float32),
    )(x2d)


def _hist_sc(xflat, wvec):
    n = xflat.shape[0]
    npw = n // _NW
    chunk = 32768
    nchunks = npw // chunk
    nvecs = chunk // _L

    mesh = plsc.VectorSubcoreMesh(core_axis_name="c", subcore_axis_name="s")

    @functools.partial(
        pl.kernel,
        out_type=jax.ShapeDtypeStruct((_NW, _NUM_BINS), jnp.float32),
        mesh=mesh,
        compiler_params=pltpu.CompilerParams(needs_layout_passes=False),
        scratch_types=[
            pltpu.VMEM((_L * _NUM_BINS,), jnp.float32),
            pltpu.VMEM((chunk,), jnp.float32),
            pltpu.VMEM((_L,), jnp.float32),
            pltpu.VMEM((_NUM_BINS,), jnp.float32),
        ],
    )
    def body(x_hbm, w_hbm, out_hbm, hist_v, buf_v, w_v, red_v):
        cid = lax.axis_index("c")
        sid = lax.axis_index("s")
        wid = cid * _NS + sid
        base = wid * npw

        pltpu.sync_copy(w_hbm, w_v)
        wv = w_v[...]
        ones = jnp.ones((_L,), jnp.float32)
        lane_base = lax.iota(jnp.int32, _L) * _NUM_BINS
        maxbin = jnp.full((_L,), _NUM_BINS - 1, jnp.int32)
        zero = jnp.zeros((_L,), jnp.float32)

        def zcol(c, carry):
            hist_v[pl.ds(c * _L, _L)] = zero
            return carry

        lax.fori_loop(0, (_L * _NUM_BINS) // _L, zcol, None)

        def chunk_body(k, carry):
            pltpu.sync_copy(x_hbm.at[pl.ds(base + k * chunk, chunk)], buf_v)

            def vec_body(j, c2):
                v = buf_v[pl.ds(j * _L, _L)]
                q = jnp.abs(v) / wv
                idx = jnp.minimum(q.astype(jnp.int32), maxbin)
                plsc.addupdate_scatter(hist_v, [lane_base + idx], ones)
                return c2

            lax.fori_loop(0, nvecs, vec_body, None)
            return carry

        lax.fori_loop(0, nchunks, chunk_body, None)

        def rcol(c, carry):
            acc = hist_v[pl.ds(c * _L, _L)]
            for r in range(1, _L):
                acc = acc + hist_v[pl.ds(r * _NUM_BINS + c * _L, _L)]
            red_v[pl.ds(c * _L, _L)] = acc
            return carry

        lax.fori_loop(0, _NUM_BINS // _L, rcol, None)
        pltpu.sync_copy(red_v, out_hbm.at[wid])

    return body(xflat, wvec)


def kernel(x):
    xf = x.astype(jnp.float32)
    x2d = xf.reshape(-1, 4096)
    x_max = _absmax(x2d)[0, 0]
    width = x_max / jnp.float32(_NUM_BINS)
    partials = _hist_sc(xf.reshape(-1), jnp.full((_L,), width))
    calib_hist = jnp.sum(partials, axis=0)
    calib_bin_edges = jnp.arange(_NUM_BINS + 1, dtype=jnp.float32) * width
    return calib_hist, calib_bin_edges


# speedup vs baseline: 197.5019x; 4.7930x over previous
import functools

import jax
import jax.numpy as jnp
from jax import lax
from jax.experimental import pallas as pl
from jax.experimental.pallas import tpu as pltpu
from jax.experimental.pallas import tpu_sc as plsc

_NUM_BINS = 2048
_L = 16
_NC = 2
_NS = 16
_NW = _NC * _NS

_COLS = 4096
_ROWS_PER_CHUNK = 8
_CHUNK = _ROWS_PER_CHUNK * _COLS
_CGROUPS = _COLS // _L

_PARAMS = pltpu.CompilerParams(
    needs_layout_passes=False, use_tc_tiling_on_sc=True)


def _absmax_sc(x2d):
    rows = x2d.shape[0]
    rows_pw = rows // _NW
    nchunks = rows_pw // _ROWS_PER_CHUNK

    mesh = plsc.VectorSubcoreMesh(core_axis_name="c", subcore_axis_name="s")

    @functools.partial(
        pl.kernel,
        out_type=jax.ShapeDtypeStruct((_NW, _L), jnp.float32),
        mesh=mesh,
        compiler_params=_PARAMS,
        scratch_types=[
            pltpu.VMEM((_ROWS_PER_CHUNK, _COLS), jnp.float32),
            pltpu.VMEM((_ROWS_PER_CHUNK, _COLS), jnp.float32),
            pltpu.VMEM((_L,), jnp.float32),
            pltpu.SemaphoreType.DMA,
            pltpu.SemaphoreType.DMA,
        ],
    )
    def body(x_hbm, out_hbm, buf0_v, buf1_v, acc_v, sem0, sem1):
        cid = lax.axis_index("c")
        sid = lax.axis_index("s")
        wid = cid * _NS + sid
        row_base = wid * rows_pw
        bufs = (buf0_v, buf1_v)
        sems = (sem0, sem1)

        def start(k, b):
            pltpu.async_copy(
                x_hbm.at[pl.ds(row_base + k * _ROWS_PER_CHUNK,
                               _ROWS_PER_CHUNK)],
                bufs[b], sems[b])

        def wait(k, b):
            pltpu.make_async_copy(
                x_hbm.at[pl.ds(row_base + k * _ROWS_PER_CHUNK,
                               _ROWS_PER_CHUNK)],
                bufs[b], sems[b]).wait()

        def compute(b, accs):
            buf = bufs[b]

            @plsc.parallel_loop(0, _CGROUPS, unroll=2, carry=accs)
            def accs_next(j, a):
                vs = [buf[r, pl.ds(j * _L, _L)]
                      for r in range(_ROWS_PER_CHUNK)]
                return tuple(
                    jnp.maximum(a[r], jnp.abs(vs[r]))
                    for r in range(_ROWS_PER_CHUNK))

            return accs_next

        z = jnp.zeros((_L,), jnp.float32)
        accs = (z,) * _ROWS_PER_CHUNK
        start(0, 0)
        start(1, 1)

        def chunk_pair(kk, a):
            for b in range(2):
                k = kk * 2 + b
                wait(k, b)
                a = compute(b, a)
                start(k + 2, b)
            return a

        accs = lax.fori_loop(0, nchunks // 2 - 1, chunk_pair, accs)
        for b in range(2):
            wait(nchunks - 2 + b, b)
            accs = compute(b, accs)

        m = accs[0]
        for r in range(1, _ROWS_PER_CHUNK):
            m = jnp.maximum(m, accs[r])
        acc_v[...] = m
        pltpu.sync_copy(acc_v, out_hbm.at[wid])

    return body(x2d)


def _hist_sc(x2d, maxes):
    rows = x2d.shape[0]
    rows_pw = rows // _NW
    nchunks = rows_pw // _ROWS_PER_CHUNK

    nslots = _NUM_BINS + 1

    mesh = plsc.VectorSubcoreMesh(core_axis_name="c", subcore_axis_name="s")

    @functools.partial(
        pl.kernel,
        out_type=jax.ShapeDtypeStruct((_NW, _NUM_BINS), jnp.float32),
        mesh=mesh,
        compiler_params=_PARAMS,
        scratch_types=[
            pltpu.VMEM((nslots * _L,), jnp.float32),
            pltpu.VMEM((_ROWS_PER_CHUNK, _COLS), jnp.float32),
            pltpu.VMEM((_ROWS_PER_CHUNK, _COLS), jnp.float32),
            pltpu.VMEM((_NW, _L), jnp.float32),
            pltpu.VMEM((_NUM_BINS,), jnp.float32),
            pltpu.SemaphoreType.DMA,
            pltpu.SemaphoreType.DMA,
        ],
    )
    def body(x_hbm, m_hbm, out_hbm, hist_v, buf0_v, buf1_v, m_v, red_v,
             sem0, sem1):
        cid = lax.axis_index("c")
        sid = lax.axis_index("s")
        wid = cid * _NS + sid
        row_base = wid * rows_pw
        bufs = (buf0_v, buf1_v)
        sems = (sem0, sem1)

        pltpu.sync_copy(m_hbm, m_v)
        m = m_v[0, :]
        for r in range(1, _NW):
            m = jnp.maximum(m, m_v[r, :])
        x_max = jnp.broadcast_to(jnp.max(m), (_L,))
        wv = x_max * jnp.float32(1.0 / _NUM_BINS)
        ones = jnp.ones((_L,), jnp.float32)
        lanes = lax.iota(jnp.int32, _L)
        zero = jnp.zeros((_L,), jnp.float32)

        @plsc.parallel_loop(0, nslots)
        def _(c):
            hist_v[pl.ds(c * _L, _L)] = zero

        def start(k, b):
            pltpu.async_copy(
                x_hbm.at[pl.ds(row_base + k * _ROWS_PER_CHUNK,
                               _ROWS_PER_CHUNK)],
                bufs[b], sems[b])

        def wait(k, b):
            pltpu.make_async_copy(
                x_hbm.at[pl.ds(row_base + k * _ROWS_PER_CHUNK,
                               _ROWS_PER_CHUNK)],
                bufs[b], sems[b]).wait()

        def compute(b):
            buf = bufs[b]

            @plsc.parallel_loop(0, _CGROUPS, unroll=2)
            def _(j):
                for r in range(_ROWS_PER_CHUNK):
                    v = buf[r, pl.ds(j * _L, _L)]
                    q = jnp.abs(v) / wv
                    flat = lax.shift_left(q.astype(jnp.int32), 4) + lanes
                    plsc.addupdate_scatter(hist_v, [flat], ones)

        start(0, 0)
        start(1, 1)

        def chunk_pair(kk, carry):
            for b in range(2):
                k = kk * 2 + b
                wait(k, b)
                compute(b)
                start(k + 2, b)
            return carry

        lax.fori_loop(0, nchunks // 2 - 1, chunk_pair, None)
        for b in range(2):
            wait(nchunks - 2 + b, b)
            compute(b)

        ngroups = _NUM_BINS // _L
        gvec = lax.iota(jnp.int32, _L) * _L

        @plsc.parallel_loop(0, ngroups)
        def _(g):
            acc = zero
            for l in range(_L):
                acc = acc + plsc.load_gather(
                    hist_v, [gvec + (g * (_L * _L) + l)])
            @pl.when(g == ngroups - 1)
            def _():
                ext = hist_v[pl.ds(_NUM_BINS * _L, _L)]
                extsum = jnp.broadcast_to(jnp.sum(ext), (_L,))
                red_v[pl.ds(g * _L, _L)] = acc + jnp.where(
                    lanes == _L - 1, extsum, zero)
            @pl.when(g != ngroups - 1)
            def _():
                red_v[pl.ds(g * _L, _L)] = acc

        pltpu.sync_copy(red_v, out_hbm.at[wid])

    return body(x2d, maxes)


def kernel(x):
    x2d = x.astype(jnp.float32).reshape(-1, _COLS)
    maxes = _absmax_sc(x2d)
    partials = _hist_sc(x2d, maxes)
    calib_hist = jnp.sum(partials, axis=0)
    x_max = jnp.max(maxes)
    width = x_max / jnp.float32(_NUM_BINS)
    calib_bin_edges = jnp.arange(_NUM_BINS + 1, dtype=jnp.float32) * width
    return calib_hist, calib_bin_edges
